# Initial kernel scaffold; baseline (speedup 1.0000x reference)
import jax
import jax.numpy as jnp
from jax.experimental import pallas as pl


def kernel(x, w_cell, b_cell, w_yvec, path, neigh_idx, neigh_valid):
    raise NotImplementedError("write your pallas kernel here")


# speedup vs baseline: 54.4161x; 54.4161x over previous
import functools

import jax
import jax.numpy as jnp
from jax import lax
from jax.experimental import pallas as pl
from jax.experimental.pallas import tpu as pltpu
from jax.experimental.pallas import tpu_sc as plsc

_B, _H, _W, _C = 128, 32, 32, 128
_HW = _H * _W
_P = 128
_LANES = 16
_NSUB = _B // _LANES


def _k1_body(x_ref, w_ref, b_ref, d1_ref, d2_ref):
    xb = x_ref[...]
    w1 = w_ref[0, :]
    w2 = w_ref[1, :]
    d1_ref[...] = jnp.sum(xb * w1[None, None, :], axis=-1) + b_ref[0, 0]
    d2_ref[...] = jnp.sum(xb * w2[None, None, :], axis=-1)


def _run_k1(xr, w12, b2):
    return pl.pallas_call(
        _k1_body,
        grid=(_HW // _P,),
        in_specs=[
            pl.BlockSpec((_B, _P, _C), lambda k: (0, k, 0)),
            pl.BlockSpec((2, _C), lambda k: (0, 0)),
            pl.BlockSpec(memory_space=pltpu.SMEM),
        ],
        out_specs=[
            pl.BlockSpec((_B, _P), lambda k: (0, k)),
            pl.BlockSpec((_B, _P), lambda k: (0, k)),
        ],
        out_shape=[
            jax.ShapeDtypeStruct((_B, _HW), jnp.float32),
            jax.ShapeDtypeStruct((_B, _HW), jnp.float32),
        ],
    )(xr, w12, b2)


_MESH = plsc.VectorSubcoreMesh(core_axis_name="c", subcore_axis_name="s")


_VS = _HW + _LANES


@functools.partial(
    pl.kernel,
    out_type=jax.ShapeDtypeStruct((_B * _HW,), jnp.float32),
    mesh=_MESH,
    compiler_params=pltpu.CompilerParams(needs_layout_passes=False),
    scratch_types=[
        pltpu.VMEM((_LANES * _HW,), jnp.float32),
        pltpu.VMEM((_LANES * _VS,), jnp.float32),
        pltpu.VMEM((_HW * _LANES,), jnp.int32),
        pltpu.VMEM((_HW * _LANES,), jnp.int32),
        pltpu.VMEM((_HW * _LANES,), jnp.int32),
        pltpu.VMEM((_HW * _LANES,), jnp.float32),
        pltpu.VMEM((2 * _C,), jnp.float32),
        pltpu.VMEM((_C,), jnp.float32),
        pltpu.SemaphoreType.DMA,
    ],
)
def _k2(d1_hbm, d2_hbm, tab_hbm, pcoly_hbm, pcolv_hbm, cnt_hbm, wc_hbm, wy_hbm,
        y_hbm, yv, vv, tabv, pcolyv, pcolvv, cntv, wcv, wyv, sem):
    wid = lax.axis_index("s") * 2 + lax.axis_index("c")

    @pl.when(wid < _NSUB)
    def _():
        base = wid * _LANES
        for r in range(_LANES):
            pltpu.sync_copy(d1_hbm.at[pl.ds((base + r) * _HW, _HW)],
                            yv.at[pl.ds(r * _HW, _HW)])
            pltpu.sync_copy(d2_hbm.at[pl.ds((base + r) * _HW, _HW)],
                            vv.at[pl.ds(r * _VS, _HW)])
        pltpu.sync_copy(tab_hbm, tabv)
        pltpu.sync_copy(pcoly_hbm, pcolyv)
        pltpu.sync_copy(pcolv_hbm, pcolvv)
        pltpu.sync_copy(cnt_hbm, cntv)
        pltpu.sync_copy(wc_hbm, wcv)
        pltpu.sync_copy(wy_hbm, wyv)

        zero = jnp.zeros((_LANES,), jnp.float32)
        rowoff = lax.broadcasted_iota(jnp.int32, (_LANES,), 0) * _VS

        plsc.store_scatter(vv, [rowoff + _HW], zero)

        sacc = zero
        for j in range(_C // _LANES):
            sacc = sacc + (wcv[pl.ds(_C + j * _LANES, _LANES)]
                           * wyv[pl.ds(j * _LANES, _LANES)])
        s = sacc[0]
        for l in range(1, _LANES):
            s = s + sacc[l]

        def step(i, carry):
            trow = tabv[pl.ds(i * _LANES, _LANES)]
            pcoly = pcolyv[pl.ds(i * _LANES, _LANES)]
            pcolv = pcolvv[pl.ds(i * _LANES, _LANES)]
            crow = cntv[pl.ds(i * _LANES, _LANES)]
            acc = zero
            for j in range(4):
                nidx = rowoff + jnp.full((_LANES,), trow[j], jnp.int32)
                acc = acc + plsc.load_gather(vv, [nidx])
            y = plsc.load_gather(yv, [pcoly]) + crow * acc
            vp = plsc.load_gather(vv, [pcolv])
            plsc.store_scatter(vv, [pcolv], vp + carry * y)
            plsc.store_scatter(yv, [pcoly], y)
            return carry

        lax.fori_loop(0, _HW, step, s)
        for r in range(_LANES):
            pltpu.sync_copy(yv.at[pl.ds(r * _HW, _HW)],
                            y_hbm.at[pl.ds((base + r) * _HW, _HW)])


def _k3_body(x_ref, y_ref, w_ref, o_ref):
    yb = y_ref[...]
    wy = w_ref[0, :]
    o_ref[...] = x_ref[...] + yb[:, :, None] * wy[None, None, :]


def _run_k3(xr, y, wy2):
    return pl.pallas_call(
        _k3_body,
        grid=(_HW // _P,),
        in_specs=[
            pl.BlockSpec((_B, _P, _C), lambda k: (0, k, 0)),
            pl.BlockSpec((_B, _P), lambda k: (0, k)),
            pl.BlockSpec((1, _C), lambda k: (0, 0)),
        ],
        out_specs=pl.BlockSpec((_B, _P, _C), lambda k: (0, k, 0)),
        out_shape=jax.ShapeDtypeStruct((_B, _HW, _C), jnp.float32),
    )(xr, y, wy2)


def kernel(x, w_cell, b_cell, w_yvec, path, neigh_idx, neigh_valid):
    xr = x.reshape(_B, _HW, _C)
    w12 = w_cell.reshape(2, _C)
    b2 = b_cell.reshape(1, 1)
    mask = neigh_valid > 0
    nidx = jnp.where(mask, neigh_idx, _HW).astype(jnp.int32)
    cntinv = 1.0 / jnp.maximum(jnp.sum(mask, axis=-1).astype(jnp.float32), 1.0)
    lanes = jnp.arange(_LANES, dtype=jnp.int32)
    p32 = path.astype(jnp.int32)
    tab = jnp.concatenate(
        [nidx, jnp.zeros((_HW, _LANES - 4), jnp.int32)], axis=1).reshape(-1)
    pcoly_t = (p32[:, None] + lanes[None, :] * _HW).reshape(-1)
    pcolv_t = (p32[:, None] + lanes[None, :] * _VS).reshape(-1)
    cnt_t = jnp.tile(cntinv[:, None], (1, _LANES)).reshape(-1)
    d1, d2 = _run_k1(xr, w12, b2)
    y = _k2(d1.reshape(-1), d2.reshape(-1), tab, pcoly_t, pcolv_t, cnt_t,
            w_cell, w_yvec)
    out = _run_k3(xr, y.reshape(_B, _HW), w_yvec.reshape(1, _C))
    return out.reshape(_B, _H, _W, _C)
